# baseline (speedup 1.0000x reference)
import functools
import math

import jax
import jax.numpy as jnp
from jax import lax
from jax.experimental import pallas as pl
from jax.experimental.pallas import tpu as pltpu
from jax.experimental.pallas import tpu_sc as plsc

D_DIM = 512
VOCAB = 256
SCALE = math.sqrt(float(D_DIM))
REPL = 8


def _make_sc_kernel(B: int):
    info = plsc.get_sparse_core_info()
    NC, NS, L = info.num_cores, info.num_subcores, info.num_lanes
    NW = NC * NS
    assert B % NW == 0
    b_per_w = B // NW
    CH = 16
    assert b_per_w % CH == 0
    n_ch = b_per_w // CH
    NBUF = 8
    rows_per_sub = VOCAB // NS

    mesh = plsc.VectorSubcoreMesh(core_axis_name="c", subcore_axis_name="s")

    @functools.partial(
        pl.kernel,
        mesh=mesh,
        out_type=jax.ShapeDtypeStruct((B, D_DIM), jnp.float32),
        scratch_types=[
            pltpu.VMEM((b_per_w,), jnp.int32),
            pltpu.VMEM((rows_per_sub, D_DIM), jnp.float32),
            pltpu.HBM((NC * REPL * VOCAB, D_DIM), jnp.float32),
            [pltpu.VMEM((CH, D_DIM), jnp.float32)] * NBUF,
            [pltpu.SemaphoreType.DMA] * NBUF,
            [pltpu.SemaphoreType.DMA] * NBUF,
            [pltpu.SemaphoreType.DMA] * REPL,
        ],
    )
    def emb_kernel(x_hbm, table_hbm, out_hbm, idx_v, tslice, stable, bufs,
                   gsems, wsems, psems):
        cid = lax.axis_index("c")
        sid = lax.axis_index("s")
        wid = sid * NC + cid

        row0 = sid * rows_per_sub
        pltpu.sync_copy(table_hbm.at[pl.ds(row0, rows_per_sub)], tslice)
        for r in range(rows_per_sub):
            def scale_body(j, carry, r=r):
                tslice[r, pl.ds(j * L, L)] = tslice[r, pl.ds(j * L, L)] * SCALE
                return carry
            lax.fori_loop(0, D_DIM // L, scale_body, 0)
        rdescs = [
            pltpu.async_copy(
                tslice,
                stable.at[pl.ds((cid * REPL + k) * VOCAB + row0, rows_per_sub)],
                psems[k],
            )
            for k in range(REPL)
        ]
        for d in rdescs:
            d.wait()
        plsc.subcore_barrier()

        base = wid * b_per_w
        pltpu.sync_copy(x_hbm.at[pl.ds(base, b_per_w)], idx_v)
        voff = (cid * REPL + sid % REPL) * VOCAB

        def bias_body(j, carry):
            idx_v[pl.ds(j * L, L)] = idx_v[pl.ds(j * L, L)] + voff
            return carry

        lax.fori_loop(0, b_per_w // L, bias_body, 0)

        def gather_start(c, b):
            return pltpu.async_copy(
                stable.at[idx_v.at[pl.ds(c * CH, CH)]], bufs[b], gsems[b]
            )

        gd = [None] * NBUF
        wd = [None] * NBUF
        pending_writes = {}
        for p in range(min(NBUF - 1, n_ch)):
            gd[p] = gather_start(p, p)
        for c in range(n_ch):
            b = c % NBUF
            gd[b].wait()
            wd[b] = pltpu.async_copy(
                bufs[b], out_hbm.at[pl.ds(base + c * CH, CH)], wsems[b]
            )
            pending_writes[b] = wd[b]
            nxt = c + NBUF - 1
            if nxt < n_ch:
                bb = nxt % NBUF
                if wd[bb] is not None:
                    wd[bb].wait()
                    pending_writes.pop(bb, None)
                gd[bb] = gather_start(nxt, bb)
        for d in pending_writes.values():
            d.wait()

    return emb_kernel


def kernel(x, table):
    B = x.shape[0] * x.shape[1]
    xf = x.reshape(B)
    out = _make_sc_kernel(B)(xf, table)
    return out.reshape(x.shape + (D_DIM,))


# speedup vs baseline: 1.0037x; 1.0037x over previous
import functools
import math

import jax
import jax.numpy as jnp
from jax import lax
from jax.experimental import pallas as pl
from jax.experimental.pallas import tpu as pltpu
from jax.experimental.pallas import tpu_sc as plsc

D_DIM = 512
VOCAB = 256
SCALE = math.sqrt(float(D_DIM))
REPL = 8


def _make_sc_kernel(B: int):
    info = plsc.get_sparse_core_info()
    NC, NS, L = info.num_cores, info.num_subcores, info.num_lanes
    NW = NC * NS
    assert B % NW == 0
    b_per_w = B // NW
    CH = 32
    assert b_per_w % CH == 0
    n_ch = b_per_w // CH
    NBUF = 7
    rows_per_sub = VOCAB // NS

    mesh = plsc.VectorSubcoreMesh(core_axis_name="c", subcore_axis_name="s")

    @functools.partial(
        pl.kernel,
        mesh=mesh,
        out_type=jax.ShapeDtypeStruct((B, D_DIM), jnp.float32),
        scratch_types=[
            pltpu.VMEM((b_per_w,), jnp.int32),
            pltpu.VMEM((rows_per_sub, D_DIM), jnp.float32),
            pltpu.HBM((NC * REPL * VOCAB, D_DIM), jnp.float32),
            [pltpu.VMEM((CH, D_DIM), jnp.float32)] * NBUF,
            [pltpu.SemaphoreType.DMA] * NBUF,
            [pltpu.SemaphoreType.DMA] * NBUF,
            [pltpu.SemaphoreType.DMA] * REPL,
        ],
    )
    def emb_kernel(x_hbm, table_hbm, out_hbm, idx_v, tslice, stable, bufs,
                   gsems, wsems, psems):
        cid = lax.axis_index("c")
        sid = lax.axis_index("s")
        wid = sid * NC + cid

        row0 = sid * rows_per_sub
        pltpu.sync_copy(table_hbm.at[pl.ds(row0, rows_per_sub)], tslice)
        for r in range(rows_per_sub):
            def scale_body(j, carry, r=r):
                tslice[r, pl.ds(j * L, L)] = tslice[r, pl.ds(j * L, L)] * SCALE
                return carry
            lax.fori_loop(0, D_DIM // L, scale_body, 0)
        rdescs = [
            pltpu.async_copy(
                tslice,
                stable.at[pl.ds((cid * REPL + k) * VOCAB + row0, rows_per_sub)],
                psems[k],
            )
            for k in range(REPL)
        ]
        for d in rdescs:
            d.wait()
        plsc.subcore_barrier()

        base = wid * b_per_w
        pltpu.sync_copy(x_hbm.at[pl.ds(base, b_per_w)], idx_v)
        voff = (cid * REPL + sid % REPL) * VOCAB

        def bias_body(j, carry):
            idx_v[pl.ds(j * L, L)] = idx_v[pl.ds(j * L, L)] + voff
            return carry

        lax.fori_loop(0, b_per_w // L, bias_body, 0)

        def gather_start(c, b):
            return pltpu.async_copy(
                stable.at[idx_v.at[pl.ds(c * CH, CH)]], bufs[b], gsems[b]
            )

        gd = [None] * NBUF
        wd = [None] * NBUF
        pending_writes = {}
        for p in range(min(NBUF - 1, n_ch)):
            gd[p] = gather_start(p, p)
        for c in range(n_ch):
            b = c % NBUF
            gd[b].wait()
            wd[b] = pltpu.async_copy(
                bufs[b], out_hbm.at[pl.ds(base + c * CH, CH)], wsems[b]
            )
            pending_writes[b] = wd[b]
            nxt = c + NBUF - 1
            if nxt < n_ch:
                bb = nxt % NBUF
                if wd[bb] is not None:
                    wd[bb].wait()
                    pending_writes.pop(bb, None)
                gd[bb] = gather_start(nxt, bb)
        for d in pending_writes.values():
            d.wait()

    return emb_kernel


def kernel(x, table):
    B = x.shape[0] * x.shape[1]
    xf = x.reshape(B)
    out = _make_sc_kernel(B)(xf, table)
    return out.reshape(x.shape + (D_DIM,))


# speedup vs baseline: 1.0225x; 1.0188x over previous
import functools
import math

import jax
import jax.numpy as jnp
from jax import lax
from jax.experimental import pallas as pl
from jax.experimental.pallas import tpu as pltpu
from jax.experimental.pallas import tpu_sc as plsc

D_DIM = 512
VOCAB = 256
SCALE = math.sqrt(float(D_DIM))
REPL = 8


def _make_sc_kernel(B: int):
    info = plsc.get_sparse_core_info()
    NC, NS, L = info.num_cores, info.num_subcores, info.num_lanes
    NW = NC * NS
    assert B % NW == 0
    b_per_w = B // NW
    CH = 32
    assert b_per_w % CH == 0
    n_ch = b_per_w // CH
    NBUF = 6
    rows_per_sub = VOCAB // NS

    mesh = plsc.VectorSubcoreMesh(core_axis_name="c", subcore_axis_name="s")

    @functools.partial(
        pl.kernel,
        mesh=mesh,
        out_type=jax.ShapeDtypeStruct((B, D_DIM), jnp.float32),
        scratch_types=[
            pltpu.VMEM((b_per_w,), jnp.int32),
            pltpu.VMEM((rows_per_sub, D_DIM), jnp.float32),
            pltpu.HBM((NC * REPL * VOCAB, D_DIM), jnp.float32),
            [pltpu.VMEM((CH, D_DIM), jnp.float32)] * NBUF,
            [pltpu.SemaphoreType.DMA] * NBUF,
            [pltpu.SemaphoreType.DMA] * NBUF,
            [pltpu.SemaphoreType.DMA] * REPL,
            pltpu.SemaphoreType.DMA,
        ],
    )
    def emb_kernel(x_hbm, table_hbm, out_hbm, idx_v, tslice, stable, bufs,
                   gsems, wsems, psems, isem):
        cid = lax.axis_index("c")
        sid = lax.axis_index("s")
        wid = sid * NC + cid
        base = wid * b_per_w

        idesc = pltpu.async_copy(x_hbm.at[pl.ds(base, b_per_w)], idx_v, isem)

        row0 = sid * rows_per_sub
        pltpu.sync_copy(table_hbm.at[pl.ds(row0, rows_per_sub)], tslice)
        for r in range(rows_per_sub):
            def scale_body(j, carry, r=r):
                tslice[r, pl.ds(j * L, L)] = tslice[r, pl.ds(j * L, L)] * SCALE
                return carry
            lax.fori_loop(0, D_DIM // L, scale_body, 0)
        rdescs = [
            pltpu.async_copy(
                tslice,
                stable.at[pl.ds((cid * REPL + k) * VOCAB + row0, rows_per_sub)],
                psems[k],
            )
            for k in range(REPL)
        ]

        idesc.wait()
        voff = (cid * REPL + sid % REPL) * VOCAB

        def bias_body(j, carry):
            idx_v[pl.ds(j * L, L)] = idx_v[pl.ds(j * L, L)] + voff
            return carry

        lax.fori_loop(0, b_per_w // L, bias_body, 0)

        for d in rdescs:
            d.wait()
        plsc.subcore_barrier()

        def gather_start(c, b):
            return pltpu.async_copy(
                stable.at[idx_v.at[pl.ds(c * CH, CH)]], bufs[b], gsems[b]
            )

        gd = [None] * NBUF
        wd = [None] * NBUF
        pending_writes = {}
        for p in range(min(NBUF - 1, n_ch)):
            gd[p] = gather_start(p, p)
        for c in range(n_ch):
            b = c % NBUF
            gd[b].wait()
            wd[b] = pltpu.async_copy(
                bufs[b], out_hbm.at[pl.ds(base + c * CH, CH)], wsems[b]
            )
            pending_writes[b] = wd[b]
            nxt = c + NBUF - 1
            if nxt < n_ch:
                bb = nxt % NBUF
                if wd[bb] is not None:
                    wd[bb].wait()
                    pending_writes.pop(bb, None)
                gd[bb] = gather_start(nxt, bb)
        for d in pending_writes.values():
            d.wait()

    return emb_kernel


def kernel(x, table):
    B = x.shape[0] * x.shape[1]
    xf = x.reshape(B)
    out = _make_sc_kernel(B)(xf, table)
    return out.reshape(x.shape + (D_DIM,))
